# baseline (speedup 1.0000x reference)
import dataclasses
import functools

import jax
import jax.numpy as jnp
from jax import lax
from jax.experimental import pallas as pl
from jax.experimental.pallas import tpu as pltpu
from jax.experimental.pallas import tpu_sc as plsc

D = 128
NC = 2
NS = 16
NW = NC * NS
CH = 128
LANES = 16

def _splat_from(ref, r):
    return plsc.load_gather(ref, [jnp.full((LANES,), r, dtype=jnp.int32)])


def _splat_ew(ed_ref, r):
    idx2 = jnp.full((LANES,), 2, dtype=jnp.int32)
    idxr = jnp.full((LANES,), r, dtype=jnp.int32)
    return plsc.bitcast(plsc.load_gather(ed_ref, [idx2, idxr]), jnp.float32)


def _sc_mesh():
    return plsc.VectorSubcoreMesh(core_axis_name="c", subcore_axis_name="s")


_SPLIT0 = 0.50


def _core_split(nchunk_balanced):
    tot = 2 * nchunk_balanced
    n0 = max(16, min(tot - 16, int(round(tot * _SPLIT0 / 4)) * 4))
    n1 = tot - n0
    assert n1 >= 16 and n0 >= 16 and n0 % 4 == 0 and n1 % 4 == 0
    return n0, n1


def _sc_cp():
    cp = pltpu.CompilerParams()
    if "needs_layout_passes" in pltpu.CompilerParams.__dataclass_fields__:
        cp = dataclasses.replace(cp, needs_layout_passes=False)
    return cp


def _zero_stripe(buf_v, acc_sh, sid, stripe):
    @pl.loop(0, CH)
    def _zero(r):
        row = buf_v.at[r]
        for q in range(D // LANES):
            row[pl.ds(q * LANES, LANES)] = jnp.zeros((LANES,), jnp.float32)

    nfull, rem = stripe // CH, stripe % CH

    @pl.loop(0, nfull)
    def _zstripe(b):
        pltpu.sync_copy(buf_v, acc_sh.at[pl.ds(sid * stripe + b * CH, CH)])

    if rem:
        pltpu.sync_copy(buf_v.at[pl.ds(0, rem)],
                        acc_sh.at[pl.ds(sid * stripe + nfull * CH, rem)])


def _deg_pass(ed, n_pad):
    ncht = ed.shape[0]
    nchunk = ncht // NW
    stripe = n_pad // NS
    assert nchunk >= 8

    @functools.partial(
        pl.kernel,
        out_type=jax.ShapeDtypeStruct((NC, n_pad, D), jnp.float32),
        mesh=_sc_mesh(),
        compiler_params=_sc_cp(),
        scratch_types=[
            pltpu.VMEM((3, CH), jnp.int32),
            pltpu.VMEM((3, CH), jnp.int32),
            pltpu.VMEM((3, CH), jnp.int32),
            pltpu.VMEM((3, CH), jnp.int32),
            pltpu.VMEM((CH, D), jnp.float32),
            pltpu.VMEM((CH, D), jnp.float32),
            pltpu.VMEM_SHARED((n_pad, D), jnp.float32),
            pltpu.SemaphoreType.DMA,
            pltpu.SemaphoreType.DMA,
            pltpu.SemaphoreType.DMA,
            pltpu.SemaphoreType.DMA,
            pltpu.SemaphoreType.DMA,
            pltpu.SemaphoreType.DMA,
        ],
    )
    def deg_kernel(ed_hbm, out_hbm, ed0, ed1, ed2, ed3, rep0, rep1, acc_sh,
                   si0, si1, si2, si3, ss0, ss1):
        eds = [ed0, ed1, ed2, ed3]
        sis = [si0, si1, si2, si3]
        reps = [rep0, rep1]
        sss = [ss0, ss1]
        cid = lax.axis_index("c")
        sid = lax.axis_index("s")
        wid = cid * NS + sid
        cbase = wid * nchunk

        _zero_stripe(rep0, acc_sh, sid, stripe)
        plsc.subcore_barrier()

        def issue_i(k, jm):
            pltpu.async_copy(ed_hbm.at[cbase + k], eds[jm % 4], sis[jm % 4])

        def wait_i(k, jm):
            pltpu.make_async_copy(ed_hbm.at[cbase + k], eds[jm % 4],
                                  sis[jm % 4]).wait()

        def issue_s(jm):
            pltpu.async_copy(reps[jm % 2], acc_sh.at[eds[jm % 4].at[1]],
                             sss[jm % 2], add=True)

        def wait_s(jm):
            pltpu.make_async_copy(reps[jm % 2], acc_sh.at[eds[jm % 4].at[1]],
                                  sss[jm % 2]).wait()

        def build(jm):
            rep, edb = reps[jm % 2], eds[jm % 4]

            @pl.loop(0, CH, step=4)
            def _rep(r0):
                for u in range(4):
                    r = r0 + u
                    w = _splat_ew(edb, r)
                    row = rep.at[r]
                    for q in range(D // LANES):
                        row[pl.ds(q * LANES, LANES)] = w

        def step(k, jm, do_wait_s, do_next_i):
            wait_i(k, jm)
            if do_wait_s:
                wait_s(jm - 2)
            build(jm)
            issue_s(jm)
            if do_next_i:
                issue_i(k + 2, jm + 2)

        issue_i(0, 0)
        issue_i(1, 1)
        step(0, 0, False, True)
        step(1, 1, False, True)

        assert nchunk % 4 == 0

        @pl.loop(2, nchunk - 2, step=4)
        def _steady(k0):
            for j2 in range(4):
                step(k0 + j2, (j2 + 2) % 4, True, True)

        step(nchunk - 2, 2, True, False)
        step(nchunk - 1, 3, True, False)
        wait_s(2)
        wait_s(3)

        plsc.subcore_barrier()
        pltpu.sync_copy(acc_sh.at[pl.ds(sid * stripe, stripe)],
                        out_hbm.at[cid, pl.ds(sid * stripe, stripe)])

    return deg_kernel(ed)


def _spmm_pass(ed, g):
    ncht = ed.shape[0]
    n_pad = g.shape[0]
    nchunk = ncht // NW
    stripe = n_pad // NS
    assert nchunk >= 16

    @functools.partial(
        pl.kernel,
        out_type=jax.ShapeDtypeStruct((NC, n_pad, D), jnp.float32),
        mesh=_sc_mesh(),
        compiler_params=_sc_cp(),
        scratch_types=(
            [pltpu.VMEM((3, CH), jnp.int32)] * 4
            + [pltpu.VMEM((CH, D), jnp.float32)] * 2
            + [pltpu.VMEM_SHARED((n_pad, D), jnp.float32)]
            + [pltpu.SemaphoreType.DMA] * 8
        ),
    )
    def spmm_kernel(ed_hbm, g_hbm, out_hbm,
                    ed0, ed1, ed2, ed3, rows0, rows1,
                    acc_sh, si0, si1, si2, si3, sg0, sg1, ss0, ss1):
        eds = [ed0, ed1, ed2, ed3]
        sis = [si0, si1, si2, si3]
        rows = [rows0, rows1]
        sgs = [sg0, sg1]
        sss = [ss0, ss1]
        cid = lax.axis_index("c")
        sid = lax.axis_index("s")

        _zero_stripe(rows0, acc_sh, sid, stripe)
        plsc.subcore_barrier()

        def pipeline(cbase, nck):
            def issue_i(k, jm):
                pltpu.async_copy(ed_hbm.at[cbase + k], eds[jm % 4], sis[jm % 4])

            def wait_i(k, jm):
                pltpu.make_async_copy(ed_hbm.at[cbase + k], eds[jm % 4],
                                      sis[jm % 4]).wait()

            def issue_g(jm):
                pltpu.async_copy(g_hbm.at[eds[jm % 4].at[0]], rows[jm % 2],
                                 sgs[jm % 2])

            def wait_g(jm):
                pltpu.make_async_copy(g_hbm.at[eds[jm % 4].at[0]],
                                      rows[jm % 2], sgs[jm % 2]).wait()

            def issue_s(jm):
                pltpu.async_copy(rows[jm % 2], acc_sh.at[eds[jm % 4].at[1]],
                                 sss[jm % 2], add=True)

            def wait_s(jm):
                pltpu.make_async_copy(rows[jm % 2],
                                      acc_sh.at[eds[jm % 4].at[1]],
                                      sss[jm % 2]).wait()

            def scale(jm):
                rws, edb = rows[jm % 2], eds[jm % 4]

                @pl.loop(0, CH, step=4)
                def _scale(r0):
                    for u in range(4):
                        r = r0 + u
                        w = _splat_ew(edb, r)
                        row = rws.at[r]
                        for q in range(D // LANES):
                            sl = pl.ds(q * LANES, LANES)
                            row[sl] = row[sl] * w

            def step(k, jm, do_wait_s, do_next_g, do_next_i):
                if do_wait_s:
                    wait_s(jm - 1)
                if do_next_g:
                    wait_i(k + 1, jm + 1)
                    issue_g(jm + 1)
                wait_g(jm)
                scale(jm)
                issue_s(jm)
                if do_next_i:
                    issue_i(k + 2, jm + 2)

            issue_i(0, 0)
            issue_i(1, 1)
            wait_i(0, 0)
            issue_g(0)
            step(0, 0, False, True, True)
            step(1, 1, True, True, True)
            step(2, 2, True, True, True)
            step(3, 3, True, True, True)

            assert nck % 4 == 0

            @pl.loop(4, nck - 4, step=4)
            def _steady(k0):
                for j in range(4):
                    step(k0 + j, j, True, True, True)

            step(nck - 4, 0, True, True, True)
            step(nck - 3, 1, True, True, True)
            step(nck - 2, 2, True, True, False)
            step(nck - 1, 3, True, False, False)
            wait_s(3)

        n0, n1 = _core_split(nchunk)

        @pl.when(cid == 0)
        def _c0():
            pipeline(sid * n0, n0)

        @pl.when(cid == 1)
        def _c1():
            pipeline(NS * n0 + sid * n1, n1)

        plsc.subcore_barrier()
        pltpu.sync_copy(acc_sh.at[pl.ds(sid * stripe, stripe)],
                        out_hbm.at[cid, pl.ds(sid * stripe, stripe)])

    return spmm_kernel(ed, g)


def _tc_grid_specs(n_pad, rows):
    grid = (n_pad // rows,)
    full = pl.BlockSpec((rows, D), lambda i: (i, 0))
    dis_s = pl.BlockSpec((rows, LANES), lambda i: (i, 0))
    acc_s = pl.BlockSpec((NC, rows, D), lambda i: (0, i, 0))
    w_s = pl.BlockSpec((D, D), lambda i: (0, 0))
    b_s = pl.BlockSpec((1, D), lambda i: (0, 0))
    return grid, full, dis_s, acc_s, w_s, b_s


def _tc_layer1(degp, x, w1, b1, rows=1024):
    n_pad = x.shape[0]
    grid, full, dis_s, acc_s, w_s, b_s = _tc_grid_specs(n_pad, rows)

    def body(degp_ref, x_ref, w_ref, b_ref, g_ref, base_ref, dis_ref):
        deg = degp_ref[0, :, 0:1] + degp_ref[1, :, 0:1] + 1.0
        d1 = lax.rsqrt(deg)
        h = jnp.dot(x_ref[...], w_ref[...],
                    preferred_element_type=jnp.float32,
                    precision=lax.Precision.HIGHEST)
        g_ref[...] = d1 * h
        base_ref[...] = (d1 * d1) * h + b_ref[...]
        dis_ref[...] = jnp.broadcast_to(d1, (d1.shape[0], LANES))

    degp_s = pl.BlockSpec((NC, rows, D), lambda i: (0, i, 0))
    return pl.pallas_call(
        body,
        grid=grid,
        in_specs=[degp_s, full, w_s, b_s],
        out_specs=[full, full, dis_s],
        out_shape=[
            jax.ShapeDtypeStruct((n_pad, D), jnp.float32),
            jax.ShapeDtypeStruct((n_pad, D), jnp.float32),
            jax.ShapeDtypeStruct((n_pad, LANES), jnp.float32),
        ],
    )(degp, x, w1, b1)


def _tc_layer2(acc, dis, base1, w2, b2, rows=1024):
    n_pad = dis.shape[0]
    grid, full, dis_s, acc_s, w_s, b_s = _tc_grid_specs(n_pad, rows)

    def body(acc_ref, dis_ref, base1_ref, w_ref, b_ref, g_ref, base2_ref):
        d1 = dis_ref[:, 0:1]
        out1 = d1 * (acc_ref[0] + acc_ref[1]) + base1_ref[...]
        h1r = jnp.maximum(out1, 0.0)
        h2 = jnp.dot(h1r, w_ref[...],
                     preferred_element_type=jnp.float32,
                     precision=lax.Precision.HIGHEST)
        g_ref[...] = d1 * h2
        base2_ref[...] = (d1 * d1) * h2 + b_ref[...]

    return pl.pallas_call(
        body,
        grid=grid,
        in_specs=[acc_s, dis_s, full, w_s, b_s],
        out_specs=[full, full],
        out_shape=[
            jax.ShapeDtypeStruct((n_pad, D), jnp.float32),
            jax.ShapeDtypeStruct((n_pad, D), jnp.float32),
        ],
    )(acc, dis, base1, w2, b2)


def _tc_final(acc, dis, base2, rows=1024):
    n_pad = dis.shape[0]
    grid, full, dis_s, acc_s, w_s, b_s = _tc_grid_specs(n_pad, rows)

    def body(acc_ref, dis_ref, base2_ref, out_ref):
        d1 = dis_ref[:, 0:1]
        out_ref[...] = d1 * (acc_ref[0] + acc_ref[1]) + base2_ref[...]

    return pl.pallas_call(
        body,
        grid=grid,
        in_specs=[acc_s, dis_s, full],
        out_specs=[full],
        out_shape=[jax.ShapeDtypeStruct((n_pad, D), jnp.float32)],
    )(acc, dis, base2)[0]


def kernel(x, edge_index, edge_weight, W1, b1, W2, b2):
    n = x.shape[0]
    e = edge_index.shape[1]

    quantum = NW * CH * 4
    e_pad = ((e + quantum - 1) // quantum) * quantum
    n_pad = ((n + NS * CH - 1) // (NS * CH)) * (NS * CH)

    src = edge_index[0].astype(jnp.int32)
    dst = edge_index[1].astype(jnp.int32)
    ew = edge_weight.astype(jnp.float32)
    if e_pad > e:
        spread = jnp.arange(e_pad - e, dtype=jnp.int32) % n
        src = jnp.concatenate([src, spread])
        dst = jnp.concatenate([dst, spread])
        ew = jnp.concatenate([ew, jnp.zeros((e_pad - e,), jnp.float32)])
    ewi = jax.lax.bitcast_convert_type(ew, jnp.int32)
    ed = (jnp.stack([src, dst, ewi])
          .reshape(3, e_pad // CH, CH).transpose(1, 0, 2))
    xp = x
    if n_pad > n:
        xp = jnp.concatenate([x, jnp.zeros((n_pad - n, D), x.dtype)], axis=0)

    b1r = b1.reshape(1, D)
    b2r = b2.reshape(1, D)

    degp = _deg_pass(ed, n_pad)
    g1, base1, dis = _tc_layer1(degp, xp, W1, b1r)
    acc1 = _spmm_pass(ed, g1)
    g2, base2 = _tc_layer2(acc1, dis, base1, W2, b2r)
    acc2 = _spmm_pass(ed, g2)
    out = _tc_final(acc2, dis, base2)
    return out[:n]


# speedup vs baseline: 1.0028x; 1.0028x over previous
import dataclasses
import functools

import jax
import jax.numpy as jnp
from jax import lax
from jax.experimental import pallas as pl
from jax.experimental.pallas import tpu as pltpu
from jax.experimental.pallas import tpu_sc as plsc

D = 128
NC = 2
NS = 16
NW = NC * NS
CH = 128
LANES = 16

def _splat_from(ref, r):
    return plsc.load_gather(ref, [jnp.full((LANES,), r, dtype=jnp.int32)])


def _splat_ew(ed_ref, r):
    idx2 = jnp.full((LANES,), 2, dtype=jnp.int32)
    idxr = jnp.full((LANES,), r, dtype=jnp.int32)
    return plsc.bitcast(plsc.load_gather(ed_ref, [idx2, idxr]), jnp.float32)


def _sc_mesh():
    return plsc.VectorSubcoreMesh(core_axis_name="c", subcore_axis_name="s")


_SPLIT0 = 0.50


def _core_split(nchunk_balanced):
    tot = 2 * nchunk_balanced
    n0 = max(16, min(tot - 16, int(round(tot * _SPLIT0 / 4)) * 4))
    n1 = tot - n0
    assert n1 >= 16 and n0 >= 16 and n0 % 4 == 0 and n1 % 4 == 0
    return n0, n1


def _sc_cp():
    cp = pltpu.CompilerParams()
    if "needs_layout_passes" in pltpu.CompilerParams.__dataclass_fields__:
        cp = dataclasses.replace(cp, needs_layout_passes=False)
    return cp


def _zero_stripe(buf_v, acc_sh, sid, stripe):
    @pl.loop(0, CH)
    def _zero(r):
        row = buf_v.at[r]
        for q in range(D // LANES):
            row[pl.ds(q * LANES, LANES)] = jnp.zeros((LANES,), jnp.float32)

    nfull, rem = stripe // CH, stripe % CH

    @pl.loop(0, nfull)
    def _zstripe(b):
        pltpu.sync_copy(buf_v, acc_sh.at[pl.ds(sid * stripe + b * CH, CH)])

    if rem:
        pltpu.sync_copy(buf_v.at[pl.ds(0, rem)],
                        acc_sh.at[pl.ds(sid * stripe + nfull * CH, rem)])


def _deg_pass(ed, n_pad):
    ncht = ed.shape[0]
    nchunk = ncht // NW
    stripe = n_pad // NS
    assert nchunk >= 8

    @functools.partial(
        pl.kernel,
        out_type=jax.ShapeDtypeStruct((NC, n_pad, D), jnp.float32),
        mesh=_sc_mesh(),
        compiler_params=_sc_cp(),
        scratch_types=[
            pltpu.VMEM((3, CH), jnp.int32),
            pltpu.VMEM((3, CH), jnp.int32),
            pltpu.VMEM((3, CH), jnp.int32),
            pltpu.VMEM((3, CH), jnp.int32),
            pltpu.VMEM((CH, D), jnp.float32),
            pltpu.VMEM((CH, D), jnp.float32),
            pltpu.VMEM_SHARED((n_pad, D), jnp.float32),
            pltpu.SemaphoreType.DMA,
            pltpu.SemaphoreType.DMA,
            pltpu.SemaphoreType.DMA,
            pltpu.SemaphoreType.DMA,
            pltpu.SemaphoreType.DMA,
            pltpu.SemaphoreType.DMA,
        ],
    )
    def deg_kernel(ed_hbm, out_hbm, ed0, ed1, ed2, ed3, rep0, rep1, acc_sh,
                   si0, si1, si2, si3, ss0, ss1):
        eds = [ed0, ed1, ed2, ed3]
        sis = [si0, si1, si2, si3]
        reps = [rep0, rep1]
        sss = [ss0, ss1]
        cid = lax.axis_index("c")
        sid = lax.axis_index("s")
        wid = cid * NS + sid
        cbase = wid * nchunk

        _zero_stripe(rep0, acc_sh, sid, stripe)
        plsc.subcore_barrier()

        def issue_i(k, jm):
            pltpu.async_copy(ed_hbm.at[cbase + k], eds[jm % 4], sis[jm % 4])

        def wait_i(k, jm):
            pltpu.make_async_copy(ed_hbm.at[cbase + k], eds[jm % 4],
                                  sis[jm % 4]).wait()

        def issue_s(jm):
            pltpu.async_copy(reps[jm % 2], acc_sh.at[eds[jm % 4].at[1]],
                             sss[jm % 2], add=True)

        def wait_s(jm):
            pltpu.make_async_copy(reps[jm % 2], acc_sh.at[eds[jm % 4].at[1]],
                                  sss[jm % 2]).wait()

        def build(jm):
            rep, edb = reps[jm % 2], eds[jm % 4]

            @pl.loop(0, CH, step=4)
            def _rep(r0):
                for u in range(4):
                    r = r0 + u
                    w = _splat_ew(edb, r)
                    row = rep.at[r]
                    for q in range(D // LANES):
                        row[pl.ds(q * LANES, LANES)] = w

        def step(k, jm, do_wait_s, do_next_i):
            wait_i(k, jm)
            if do_wait_s:
                wait_s(jm - 2)
            build(jm)
            issue_s(jm)
            if do_next_i:
                issue_i(k + 2, jm + 2)

        issue_i(0, 0)
        issue_i(1, 1)
        step(0, 0, False, True)
        step(1, 1, False, True)

        assert nchunk % 4 == 0

        @pl.loop(2, nchunk - 2, step=4)
        def _steady(k0):
            for j2 in range(4):
                step(k0 + j2, (j2 + 2) % 4, True, True)

        step(nchunk - 2, 2, True, False)
        step(nchunk - 1, 3, True, False)
        wait_s(2)
        wait_s(3)

        plsc.subcore_barrier()
        pltpu.sync_copy(acc_sh.at[pl.ds(sid * stripe, stripe)],
                        out_hbm.at[cid, pl.ds(sid * stripe, stripe)])

    return deg_kernel(ed)


def _spmm_pass(ed, g):
    ncht = ed.shape[0]
    n_pad = g.shape[0]
    nchunk = ncht // NW
    stripe = n_pad // NS
    assert nchunk >= 16

    @functools.partial(
        pl.kernel,
        out_type=jax.ShapeDtypeStruct((NC, n_pad, D), jnp.float32),
        mesh=_sc_mesh(),
        compiler_params=_sc_cp(),
        scratch_types=(
            [pltpu.VMEM((3, CH), jnp.int32)] * 4
            + [pltpu.VMEM((CH, D), jnp.float32)] * 2
            + [pltpu.VMEM_SHARED((n_pad, D), jnp.float32)]
            + [pltpu.SemaphoreType.DMA] * 8
        ),
    )
    def spmm_kernel(ed_hbm, g_hbm, out_hbm,
                    ed0, ed1, ed2, ed3, rows0, rows1,
                    acc_sh, si0, si1, si2, si3, sg0, sg1, ss0, ss1):
        eds = [ed0, ed1, ed2, ed3]
        sis = [si0, si1, si2, si3]
        rows = [rows0, rows1]
        sgs = [sg0, sg1]
        sss = [ss0, ss1]
        cid = lax.axis_index("c")
        sid = lax.axis_index("s")

        _zero_stripe(rows0, acc_sh, sid, stripe)
        plsc.subcore_barrier()

        def pipeline(cbase, nck):
            def issue_i(k, jm):
                pltpu.async_copy(ed_hbm.at[cbase + k], eds[jm % 4], sis[jm % 4])

            def wait_i(k, jm):
                pltpu.make_async_copy(ed_hbm.at[cbase + k], eds[jm % 4],
                                      sis[jm % 4]).wait()

            def issue_g(jm):
                pltpu.async_copy(g_hbm.at[eds[jm % 4].at[0]], rows[jm % 2],
                                 sgs[jm % 2])

            def wait_g(jm):
                pltpu.make_async_copy(g_hbm.at[eds[jm % 4].at[0]],
                                      rows[jm % 2], sgs[jm % 2]).wait()

            def issue_s(jm):
                pltpu.async_copy(rows[jm % 2], acc_sh.at[eds[jm % 4].at[1]],
                                 sss[jm % 2], add=True)

            def wait_s(jm):
                pltpu.make_async_copy(rows[jm % 2],
                                      acc_sh.at[eds[jm % 4].at[1]],
                                      sss[jm % 2]).wait()

            def scale(jm):
                rws, edb = rows[jm % 2], eds[jm % 4]

                @pl.loop(0, CH, step=4)
                def _scale(r0):
                    for u in range(4):
                        r = r0 + u
                        w = _splat_ew(edb, r)
                        row = rws.at[r]
                        for q in range(D // LANES):
                            sl = pl.ds(q * LANES, LANES)
                            row[sl] = row[sl] * w

            def step(k, jm, do_wait_s, do_next_g, do_next_i):
                if do_wait_s:
                    wait_s(jm - 1)
                if do_next_g:
                    wait_i(k + 1, jm + 1)
                    issue_g(jm + 1)
                wait_g(jm)
                scale(jm)
                issue_s(jm)
                if do_next_i:
                    issue_i(k + 2, jm + 2)

            issue_i(0, 0)
            issue_i(1, 1)
            wait_i(0, 0)
            issue_g(0)
            step(0, 0, False, True, True)
            step(1, 1, True, True, True)
            step(2, 2, True, True, True)
            step(3, 3, True, True, True)

            assert nck % 4 == 0

            @pl.loop(4, nck - 4, step=4)
            def _steady(k0):
                for j in range(4):
                    step(k0 + j, j, True, True, True)

            step(nck - 4, 0, True, True, True)
            step(nck - 3, 1, True, True, True)
            step(nck - 2, 2, True, True, False)
            step(nck - 1, 3, True, False, False)
            wait_s(3)

        n0, n1 = _core_split(nchunk)

        @pl.when(cid == 0)
        def _c0():
            pipeline(sid * n0, n0)

        @pl.when(cid == 1)
        def _c1():
            pipeline(NS * n0 + sid * n1, n1)

        plsc.subcore_barrier()
        pltpu.sync_copy(acc_sh.at[pl.ds(sid * stripe, stripe)],
                        out_hbm.at[cid, pl.ds(sid * stripe, stripe)])

    return spmm_kernel(ed, g)


def _tc_grid_specs(n_pad, rows):
    grid = (n_pad // rows,)
    full = pl.BlockSpec((rows, D), lambda i: (i, 0))
    dis_s = pl.BlockSpec((rows, LANES), lambda i: (i, 0))
    acc_s = pl.BlockSpec((NC, rows, D), lambda i: (0, i, 0))
    w_s = pl.BlockSpec((D, D), lambda i: (0, 0))
    b_s = pl.BlockSpec((1, D), lambda i: (0, 0))
    return grid, full, dis_s, acc_s, w_s, b_s


def _tc_matmul(x, w, rows=1024):
    n_pad = x.shape[0]
    grid, full, dis_s, acc_s, w_s, b_s = _tc_grid_specs(n_pad, rows)

    def body(x_ref, w_ref, h_ref):
        h_ref[...] = jnp.dot(x_ref[...], w_ref[...],
                             preferred_element_type=jnp.float32,
                             precision=lax.Precision.HIGHEST)

    return pl.pallas_call(
        body,
        grid=grid,
        in_specs=[full, w_s],
        out_specs=[full],
        out_shape=[jax.ShapeDtypeStruct((n_pad, D), jnp.float32)],
    )(x, w)[0]


def _tc_layer1(degp, h, b1, rows=1024):
    n_pad = h.shape[0]
    grid, full, dis_s, acc_s, w_s, b_s = _tc_grid_specs(n_pad, rows)

    def body(degp_ref, h_ref, b_ref, g_ref, base_ref, dis_ref):
        deg = degp_ref[0, :, 0:1] + degp_ref[1, :, 0:1] + 1.0
        d1 = lax.rsqrt(deg)
        h_blk = h_ref[...]
        g_ref[...] = d1 * h_blk
        base_ref[...] = (d1 * d1) * h_blk + b_ref[...]
        dis_ref[...] = jnp.broadcast_to(d1, (d1.shape[0], LANES))

    degp_s = pl.BlockSpec((NC, rows, D), lambda i: (0, i, 0))
    return pl.pallas_call(
        body,
        grid=grid,
        in_specs=[degp_s, full, b_s],
        out_specs=[full, full, dis_s],
        out_shape=[
            jax.ShapeDtypeStruct((n_pad, D), jnp.float32),
            jax.ShapeDtypeStruct((n_pad, D), jnp.float32),
            jax.ShapeDtypeStruct((n_pad, LANES), jnp.float32),
        ],
    )(degp, h, b1)


def _tc_layer2(acc, dis, base1, w2, b2, rows=1024):
    n_pad = dis.shape[0]
    grid, full, dis_s, acc_s, w_s, b_s = _tc_grid_specs(n_pad, rows)

    def body(acc_ref, dis_ref, base1_ref, w_ref, b_ref, g_ref, base2_ref):
        d1 = dis_ref[:, 0:1]
        out1 = d1 * (acc_ref[0] + acc_ref[1]) + base1_ref[...]
        h1r = jnp.maximum(out1, 0.0)
        h2 = jnp.dot(h1r, w_ref[...],
                     preferred_element_type=jnp.float32,
                     precision=lax.Precision.HIGHEST)
        g_ref[...] = d1 * h2
        base2_ref[...] = (d1 * d1) * h2 + b_ref[...]

    return pl.pallas_call(
        body,
        grid=grid,
        in_specs=[acc_s, dis_s, full, w_s, b_s],
        out_specs=[full, full],
        out_shape=[
            jax.ShapeDtypeStruct((n_pad, D), jnp.float32),
            jax.ShapeDtypeStruct((n_pad, D), jnp.float32),
        ],
    )(acc, dis, base1, w2, b2)


def _tc_final(acc, dis, base2, rows=1024):
    n_pad = dis.shape[0]
    grid, full, dis_s, acc_s, w_s, b_s = _tc_grid_specs(n_pad, rows)

    def body(acc_ref, dis_ref, base2_ref, out_ref):
        d1 = dis_ref[:, 0:1]
        out_ref[...] = d1 * (acc_ref[0] + acc_ref[1]) + base2_ref[...]

    return pl.pallas_call(
        body,
        grid=grid,
        in_specs=[acc_s, dis_s, full],
        out_specs=[full],
        out_shape=[jax.ShapeDtypeStruct((n_pad, D), jnp.float32)],
    )(acc, dis, base2)[0]


def kernel(x, edge_index, edge_weight, W1, b1, W2, b2):
    n = x.shape[0]
    e = edge_index.shape[1]

    quantum = NW * CH * 4
    e_pad = ((e + quantum - 1) // quantum) * quantum
    n_pad = ((n + NS * CH - 1) // (NS * CH)) * (NS * CH)

    src = edge_index[0].astype(jnp.int32)
    dst = edge_index[1].astype(jnp.int32)
    ew = edge_weight.astype(jnp.float32)
    if e_pad > e:
        spread = jnp.arange(e_pad - e, dtype=jnp.int32) % n
        src = jnp.concatenate([src, spread])
        dst = jnp.concatenate([dst, spread])
        ew = jnp.concatenate([ew, jnp.zeros((e_pad - e,), jnp.float32)])
    ewi = jax.lax.bitcast_convert_type(ew, jnp.int32)
    ed = (jnp.stack([src, dst, ewi])
          .reshape(3, e_pad // CH, CH).transpose(1, 0, 2))
    xp = x
    if n_pad > n:
        xp = jnp.concatenate([x, jnp.zeros((n_pad - n, D), x.dtype)], axis=0)

    b1r = b1.reshape(1, D)
    b2r = b2.reshape(1, D)

    degp = _deg_pass(ed, n_pad)
    h1 = _tc_matmul(xp, W1)
    g1, base1, dis = _tc_layer1(degp, h1, b1r)
    acc1 = _spmm_pass(ed, g1)
    g2, base2 = _tc_layer2(acc1, dis, base1, W2, b2r)
    acc2 = _spmm_pass(ed, g2)
    out = _tc_final(acc2, dis, base2)
    return out[:n]
